# Initial kernel scaffold; baseline (speedup 1.0000x reference)
import jax
import jax.numpy as jnp
from jax.experimental import pallas as pl


def kernel(x, training, idx_a_0, idx_b_0, w_0, idx_a_1, idx_b_1, w_1, idx_a_2, idx_b_2, w_2, idx_a_3, idx_b_3, w_3):
    raise NotImplementedError("write your pallas kernel here")


# speedup vs baseline: 4.5983x; 4.5983x over previous
import functools

import numpy as np
import jax
import jax.numpy as jnp
from jax import lax
from jax.experimental import pallas as pl
from jax.experimental.pallas import tpu as pltpu
from jax.experimental.pallas import tpu_sc as plsc

OUT_DIM = 16000
NUM_CLASSES = 10
TAU = 10.0
BATCH = 256

NC, NS, L = 2, 16, 16
NW = NC * NS
CH = 64
NCHUNK = OUT_DIM // CH
TPW = (NCHUNK + NW - 1) // NW

_C = np.array(
    [
        [0.0, 0.0, 0.0, 0.0],
        [0.0, 0.0, 0.0, 1.0],
        [0.0, 1.0, 0.0, -1.0],
        [0.0, 1.0, 0.0, 0.0],
        [0.0, 0.0, 1.0, -1.0],
        [0.0, 0.0, 1.0, 0.0],
        [0.0, 1.0, 1.0, -2.0],
        [0.0, 1.0, 1.0, -1.0],
        [1.0, -1.0, -1.0, 1.0],
        [1.0, -1.0, -1.0, 2.0],
        [1.0, 0.0, -1.0, 0.0],
        [1.0, 0.0, -1.0, 1.0],
        [1.0, -1.0, 0.0, 0.0],
        [1.0, -1.0, 0.0, 1.0],
        [1.0, 0.0, 0.0, -1.0],
        [1.0, 0.0, 0.0, 0.0],
    ],
    dtype=np.float32,
)
_CB = np.repeat(_C, L, axis=1)


def _coeff_body(w_ref, t_ref, cb_ref, o_ref):
    w = w_ref[...]
    m = jnp.max(w, axis=1, keepdims=True)
    e = jnp.exp(w - m)
    p_train = e / jnp.sum(e, axis=1, keepdims=True)
    iota = lax.broadcasted_iota(jnp.int32, w.shape, 1)
    am = jnp.min(jnp.where(w == m, iota, 16), axis=1, keepdims=True)
    p_eval = (iota == am).astype(jnp.float32)
    probs = jnp.where(t_ref[0, 0] != 0.0, p_train, p_eval)
    o_ref[...] = jnp.dot(probs, cb_ref[...], preferred_element_type=jnp.float32)


def _coeffs(ws, training):
    w_all = jnp.concatenate(ws, axis=0)
    t = jnp.asarray(training, jnp.float32).reshape(1, 1)
    bs = 2000
    pexp = pl.pallas_call(
        _coeff_body,
        grid=(w_all.shape[0] // bs,),
        in_specs=[
            pl.BlockSpec((bs, 16), lambda i: (i, 0)),
            pl.BlockSpec((1, 1), lambda i: (0, 0)),
            pl.BlockSpec((16, 4 * L), lambda i: (0, 0)),
        ],
        out_specs=pl.BlockSpec((bs, 4 * L), lambda i: (i, 0)),
        out_shape=jax.ShapeDtypeStruct((w_all.shape[0], 4 * L), jnp.float32),
    )(w_all, t, jnp.asarray(_CB))
    n = ws[0].shape[0]
    return [pexp[l * n:(l + 1) * n] for l in range(len(ws))]


def _sc_layer(x_t, idx_a, idx_b, pexp):
    mesh = plsc.VectorSubcoreMesh(core_axis_name="c", subcore_axis_name="s")

    @functools.partial(
        pl.kernel,
        out_type=jax.ShapeDtypeStruct((OUT_DIM, BATCH), jnp.float32),
        mesh=mesh,
        scratch_types=[
            pltpu.VMEM((CH,), jnp.int32),
            pltpu.VMEM((CH,), jnp.int32),
            pltpu.VMEM((CH, BATCH), jnp.float32),
            pltpu.VMEM((CH, BATCH), jnp.float32),
            pltpu.VMEM((CH, BATCH), jnp.float32),
            pltpu.VMEM((CH, 4 * L), jnp.float32),
            pltpu.SemaphoreType.DMA,
            pltpu.SemaphoreType.DMA,
        ],
    )
    def layer(x_hbm, ia_hbm, ib_hbm, p_hbm, out_hbm,
              ia_v, ib_v, ra_v, rb_v, o_v, p_v, sem_a, sem_b):
        wid = lax.axis_index("s") * NC + lax.axis_index("c")

        def chunk_body(t, carry):
            c = t * NW + wid

            @pl.when(c < NCHUNK)
            def _():
                base = c * CH
                pltpu.sync_copy(ia_hbm.at[pl.ds(base, CH)], ia_v)
                pltpu.sync_copy(ib_hbm.at[pl.ds(base, CH)], ib_v)
                pltpu.sync_copy(p_hbm.at[pl.ds(base, CH)], p_v)
                cp_a = pltpu.async_copy(x_hbm.at[ia_v], ra_v, sem_a)
                cp_b = pltpu.async_copy(x_hbm.at[ib_v], rb_v, sem_b)
                cp_a.wait()
                cp_b.wait()

                def neuron(i, carry2):
                    p0 = p_v[i, pl.ds(0, L)]
                    p1 = p_v[i, pl.ds(L, L)]
                    p2 = p_v[i, pl.ds(2 * L, L)]
                    p3 = p_v[i, pl.ds(3 * L, L)]
                    for j in range(BATCH // L):
                        a = ra_v[i, pl.ds(j * L, L)]
                        b = rb_v[i, pl.ds(j * L, L)]
                        o_v[i, pl.ds(j * L, L)] = p0 + p1 * a + p2 * b + p3 * (a * b)
                    return carry2

                lax.fori_loop(0, CH, neuron, 0)
                pltpu.sync_copy(o_v, out_hbm.at[pl.ds(base, CH)])

            return carry

        lax.fori_loop(0, TPW, chunk_body, 0)

    return layer(x_t, idx_a, idx_b, pexp)


def _gsum_body(y_ref, o_ref):
    gsz = OUT_DIM // NUM_CLASSES
    rows = [jnp.sum(y_ref[pl.ds(g * gsz, gsz), :], axis=0)
            for g in range(NUM_CLASSES)]
    o_ref[...] = jnp.stack(rows, axis=0) / TAU


def _group_sum(y_t):
    return pl.pallas_call(
        _gsum_body,
        out_shape=jax.ShapeDtypeStruct((NUM_CLASSES, BATCH), jnp.float32),
    )(y_t)


def kernel(x, training, idx_a_0, idx_b_0, w_0, idx_a_1, idx_b_1, w_1,
           idx_a_2, idx_b_2, w_2, idx_a_3, idx_b_3, w_3):
    x = x.reshape((x.shape[0], -1))
    pexps = _coeffs([w_0, w_1, w_2, w_3], training)
    idx_as = [idx_a_0, idx_a_1, idx_a_2, idx_a_3]
    idx_bs = [idx_b_0, idx_b_1, idx_b_2, idx_b_3]
    h = x.T
    for l in range(4):
        h = _sc_layer(h, idx_as[l].astype(jnp.int32), idx_bs[l].astype(jnp.int32),
                      pexps[l])
    return _group_sum(h).T


# speedup vs baseline: 6.6035x; 1.4361x over previous
import functools

import numpy as np
import jax
import jax.numpy as jnp
from jax import lax
from jax.experimental import pallas as pl
from jax.experimental.pallas import tpu as pltpu
from jax.experimental.pallas import tpu_sc as plsc

OUT_DIM = 16000
NUM_CLASSES = 10
TAU = 10.0
BATCH = 256

NC, NS, L = 2, 16, 16
NW = NC * NS
CH = 64
NCHUNK = OUT_DIM // CH
TPW = (NCHUNK + NW - 1) // NW
PAD_OUT = NW * TPW * CH

_C = np.array(
    [
        [0.0, 0.0, 0.0, 0.0],
        [0.0, 0.0, 0.0, 1.0],
        [0.0, 1.0, 0.0, -1.0],
        [0.0, 1.0, 0.0, 0.0],
        [0.0, 0.0, 1.0, -1.0],
        [0.0, 0.0, 1.0, 0.0],
        [0.0, 1.0, 1.0, -2.0],
        [0.0, 1.0, 1.0, -1.0],
        [1.0, -1.0, -1.0, 1.0],
        [1.0, -1.0, -1.0, 2.0],
        [1.0, 0.0, -1.0, 0.0],
        [1.0, 0.0, -1.0, 1.0],
        [1.0, -1.0, 0.0, 0.0],
        [1.0, -1.0, 0.0, 1.0],
        [1.0, 0.0, 0.0, -1.0],
        [1.0, 0.0, 0.0, 0.0],
    ],
    dtype=np.float32,
)
_CB = np.repeat(_C, L, axis=1)


def _coeff_body(w_ref, t_ref, cb_ref, o_ref):
    w = w_ref[...]
    m = jnp.max(w, axis=1, keepdims=True)
    e = jnp.exp(w - m)
    p_train = e / jnp.sum(e, axis=1, keepdims=True)
    iota = lax.broadcasted_iota(jnp.int32, w.shape, 1)
    am = jnp.min(jnp.where(w == m, iota, 16), axis=1, keepdims=True)
    p_eval = (iota == am).astype(jnp.float32)
    probs = jnp.where(t_ref[0, 0] != 0.0, p_train, p_eval)
    o_ref[...] = jnp.dot(probs, cb_ref[...], preferred_element_type=jnp.float32)


def _coeffs(ws, training):
    w_all = jnp.concatenate(ws, axis=0)
    t = jnp.asarray(training, jnp.float32).reshape(1, 1)
    bs = 2000
    pexp = pl.pallas_call(
        _coeff_body,
        grid=(w_all.shape[0] // bs,),
        in_specs=[
            pl.BlockSpec((bs, 16), lambda i: (i, 0)),
            pl.BlockSpec((1, 1), lambda i: (0, 0)),
            pl.BlockSpec((16, 4 * L), lambda i: (0, 0)),
        ],
        out_specs=pl.BlockSpec((bs, 4 * L), lambda i: (i, 0)),
        out_shape=jax.ShapeDtypeStruct((w_all.shape[0], 4 * L), jnp.float32),
    )(w_all, t, jnp.asarray(_CB))
    n = ws[0].shape[0]
    return [pexp[l * n:(l + 1) * n] for l in range(len(ws))]


def _sc_layer(x_t, idx_a, idx_b, pexp):
    mesh = plsc.VectorSubcoreMesh(core_axis_name="c", subcore_axis_name="s")

    @functools.partial(
        pl.kernel,
        out_type=jax.ShapeDtypeStruct((PAD_OUT, BATCH), jnp.float32),
        mesh=mesh,
        scratch_types=[
            pltpu.VMEM((TPW, CH), jnp.int32),
            pltpu.VMEM((TPW, CH), jnp.int32),
            pltpu.VMEM((2, CH, BATCH), jnp.float32),
            pltpu.VMEM((2, CH, BATCH), jnp.float32),
            pltpu.VMEM((2, CH, BATCH), jnp.float32),
            pltpu.VMEM((2, CH, 4 * L), jnp.float32),
            pltpu.SemaphoreType.DMA,
            pltpu.SemaphoreType.DMA,
            pltpu.SemaphoreType.DMA,
            pltpu.SemaphoreType.DMA,
            pltpu.SemaphoreType.DMA,
        ],
    )
    def layer(x_hbm, ia_hbm, ib_hbm, p_hbm, out_hbm,
              ia_all, ib_all, ra_v, rb_v, o_v, p_v,
              sem_idx, sem_in0, sem_in1, sem_out0, sem_out1):
        wid = lax.axis_index("s") * NC + lax.axis_index("c")
        rbase = [jnp.minimum(t * NW + wid, NCHUNK - 1) * CH for t in range(TPW)]
        wbase = [(t * NW + wid) * CH for t in range(TPW)]
        sem_in = [sem_in0, sem_in1]
        sem_out = [sem_out0, sem_out1]

        idx_descs = []
        for t in range(TPW):
            idx_descs.append(
                pltpu.async_copy(ia_hbm.at[pl.ds(rbase[t], CH)], ia_all.at[t], sem_idx))
            idx_descs.append(
                pltpu.async_copy(ib_hbm.at[pl.ds(rbase[t], CH)], ib_all.at[t], sem_idx))
        for dsc in idx_descs:
            dsc.wait()

        in_descs = [None] * TPW
        out_descs = [None] * TPW

        def issue(t):
            b = t % 2
            in_descs[t] = (
                pltpu.async_copy(x_hbm.at[ia_all.at[t]], ra_v.at[b], sem_in[b]),
                pltpu.async_copy(x_hbm.at[ib_all.at[t]], rb_v.at[b], sem_in[b]),
                pltpu.async_copy(p_hbm.at[pl.ds(rbase[t], CH)], p_v.at[b], sem_in[b]),
            )

        issue(0)
        for t in range(TPW):
            b = t % 2
            for dsc in in_descs[t]:
                dsc.wait()
            if t + 1 < TPW:
                issue(t + 1)
            if t >= 2:
                out_descs[t - 2].wait()

            def neuron(i, carry):
                p0 = p_v[b, i, pl.ds(0, L)]
                p1 = p_v[b, i, pl.ds(L, L)]
                p2 = p_v[b, i, pl.ds(2 * L, L)]
                p3 = p_v[b, i, pl.ds(3 * L, L)]
                for j in range(BATCH // L):
                    a = ra_v[b, i, pl.ds(j * L, L)]
                    bb = rb_v[b, i, pl.ds(j * L, L)]
                    o_v[b, i, pl.ds(j * L, L)] = p0 + p1 * a + p2 * bb + p3 * (a * bb)
                return carry

            lax.fori_loop(0, CH, neuron, 0)
            out_descs[t] = pltpu.async_copy(
                o_v.at[b], out_hbm.at[pl.ds(wbase[t], CH)], sem_out[b])
        out_descs[TPW - 2].wait()
        out_descs[TPW - 1].wait()

    return layer(x_t, idx_a, idx_b, pexp)


def _gsum_body(y_ref, o_ref):
    gsz = OUT_DIM // NUM_CLASSES
    rows = [jnp.sum(y_ref[pl.ds(g * gsz, gsz), :], axis=0)
            for g in range(NUM_CLASSES)]
    o_ref[...] = jnp.stack(rows, axis=0) / TAU


def _group_sum(y_t):
    return pl.pallas_call(
        _gsum_body,
        out_shape=jax.ShapeDtypeStruct((NUM_CLASSES, BATCH), jnp.float32),
    )(y_t)


def kernel(x, training, idx_a_0, idx_b_0, w_0, idx_a_1, idx_b_1, w_1,
           idx_a_2, idx_b_2, w_2, idx_a_3, idx_b_3, w_3):
    x = x.reshape((x.shape[0], -1))
    pexps = _coeffs([w_0, w_1, w_2, w_3], training)
    idx_as = [idx_a_0, idx_a_1, idx_a_2, idx_a_3]
    idx_bs = [idx_b_0, idx_b_1, idx_b_2, idx_b_3]
    h = x.T
    for l in range(4):
        h = _sc_layer(h, idx_as[l].astype(jnp.int32), idx_bs[l].astype(jnp.int32),
                      pexps[l])
    return _group_sum(h).T


# speedup vs baseline: 7.7148x; 1.1683x over previous
import functools

import numpy as np
import jax
import jax.numpy as jnp
from jax import lax
from jax.experimental import pallas as pl
from jax.experimental.pallas import tpu as pltpu
from jax.experimental.pallas import tpu_sc as plsc

OUT_DIM = 16000
NUM_CLASSES = 10
TAU = 10.0
BATCH = 256

NC, NS, L = 2, 16, 16
NW = NC * NS
CH = 64
NCHUNK = OUT_DIM // CH
TPW = (NCHUNK + NW - 1) // NW
PAD_OUT = NW * TPW * CH

_C = np.array(
    [
        [0.0, 0.0, 0.0, 0.0],
        [0.0, 0.0, 0.0, 1.0],
        [0.0, 1.0, 0.0, -1.0],
        [0.0, 1.0, 0.0, 0.0],
        [0.0, 0.0, 1.0, -1.0],
        [0.0, 0.0, 1.0, 0.0],
        [0.0, 1.0, 1.0, -2.0],
        [0.0, 1.0, 1.0, -1.0],
        [1.0, -1.0, -1.0, 1.0],
        [1.0, -1.0, -1.0, 2.0],
        [1.0, 0.0, -1.0, 0.0],
        [1.0, 0.0, -1.0, 1.0],
        [1.0, -1.0, 0.0, 0.0],
        [1.0, -1.0, 0.0, 1.0],
        [1.0, 0.0, 0.0, -1.0],
        [1.0, 0.0, 0.0, 0.0],
    ],
    dtype=np.float32,
)
_CB = np.repeat(_C, L, axis=1)


def _coeff_body(x_ref, w0_ref, w1_ref, w2_ref, w3_ref, t_ref, cb_ref,
                xt_ref, o0_ref, o1_ref, o2_ref, o3_ref):
    xt_ref[...] = x_ref[...].T

    def one(w):
        m = jnp.max(w, axis=1, keepdims=True)
        e = jnp.exp(w - m)
        p_train = e / jnp.sum(e, axis=1, keepdims=True)
        iota = lax.broadcasted_iota(jnp.int32, w.shape, 1)
        am = jnp.min(jnp.where(w == m, iota, 16), axis=1, keepdims=True)
        p_eval = (iota == am).astype(jnp.float32)
        probs = jnp.where(t_ref[0, 0] != 0.0, p_train, p_eval)
        return jnp.dot(probs, cb_ref[...], preferred_element_type=jnp.float32)

    o0_ref[...] = one(w0_ref[...])
    o1_ref[...] = one(w1_ref[...])
    o2_ref[...] = one(w2_ref[...])
    o3_ref[...] = one(w3_ref[...])


def _coeffs(x, ws, training):
    n = ws[0].shape[0]
    in_d = x.shape[1]
    t = jnp.asarray(training, jnp.float32).reshape(1, 1)
    grid = 8
    bs = n // grid
    cs = in_d // grid
    w_spec = pl.BlockSpec((bs, 16), lambda i: (i, 0))
    o_spec = pl.BlockSpec((bs, 4 * L), lambda i: (i, 0))
    outs = pl.pallas_call(
        _coeff_body,
        grid=(grid,),
        in_specs=[pl.BlockSpec((x.shape[0], cs), lambda i: (0, i)),
                  w_spec, w_spec, w_spec, w_spec,
                  pl.BlockSpec((1, 1), lambda i: (0, 0)),
                  pl.BlockSpec((16, 4 * L), lambda i: (0, 0))],
        out_specs=[pl.BlockSpec((cs, x.shape[0]), lambda i: (i, 0)),
                   o_spec, o_spec, o_spec, o_spec],
        out_shape=[jax.ShapeDtypeStruct((in_d, x.shape[0]), jnp.float32)] +
                  [jax.ShapeDtypeStruct((n, 4 * L), jnp.float32)] * 4,
    )(x, *ws, t, jnp.asarray(_CB))
    return outs[0], list(outs[1:])


def _sc_layer_gsum(x_t, idx_a, idx_b, pexp):
    mesh = plsc.VectorSubcoreMesh(core_axis_name="c", subcore_axis_name="s")

    @functools.partial(
        pl.kernel,
        out_type=jax.ShapeDtypeStruct((NW * TPW, BATCH), jnp.float32),
        mesh=mesh,
        scratch_types=[
            pltpu.VMEM((TPW, CH), jnp.int32),
            pltpu.VMEM((TPW, CH), jnp.int32),
            pltpu.VMEM((2, CH, BATCH), jnp.float32),
            pltpu.VMEM((2, CH, BATCH), jnp.float32),
            pltpu.VMEM((TPW, BATCH), jnp.float32),
            pltpu.VMEM((2, CH, 4 * L), jnp.float32),
            pltpu.SemaphoreType.DMA,
            pltpu.SemaphoreType.DMA,
            pltpu.SemaphoreType.DMA,
            pltpu.SemaphoreType.DMA,
        ],
    )
    def layer(x_hbm, ia_hbm, ib_hbm, p_hbm, out_hbm,
              ia_all, ib_all, ra_v, rb_v, part_v, p_v,
              sem_idx, sem_in0, sem_in1, sem_out):
        wid = lax.axis_index("s") * NC + lax.axis_index("c")
        rbase = [jnp.minimum(t * NW + wid, NCHUNK - 1) * CH for t in range(TPW)]
        wrow = [t * NW + wid for t in range(TPW)]
        sem_in = [sem_in0, sem_in1]

        idx_descs = []
        for t in range(TPW):
            idx_descs.append(
                pltpu.async_copy(ia_hbm.at[pl.ds(rbase[t], CH)], ia_all.at[t], sem_idx))
            idx_descs.append(
                pltpu.async_copy(ib_hbm.at[pl.ds(rbase[t], CH)], ib_all.at[t], sem_idx))
        for dsc in idx_descs:
            dsc.wait()

        in_descs = [None] * TPW
        out_descs = []

        def issue(t):
            b = t % 2
            in_descs[t] = (
                pltpu.async_copy(x_hbm.at[ia_all.at[t]], ra_v.at[b], sem_in[b]),
                pltpu.async_copy(x_hbm.at[ib_all.at[t]], rb_v.at[b], sem_in[b]),
                pltpu.async_copy(p_hbm.at[pl.ds(rbase[t], CH)], p_v.at[b], sem_in[b]),
            )

        issue(0)
        for t in range(TPW):
            b = t % 2
            for dsc in in_descs[t]:
                dsc.wait()
            if t + 1 < TPW:
                issue(t + 1)

            def neuron(i, accs):
                p0 = p_v[b, i, pl.ds(0, L)]
                p1 = p_v[b, i, pl.ds(L, L)]
                p2 = p_v[b, i, pl.ds(2 * L, L)]
                p3 = p_v[b, i, pl.ds(3 * L, L)]
                out = []
                for j in range(BATCH // L):
                    a = ra_v[b, i, pl.ds(j * L, L)]
                    bb = rb_v[b, i, pl.ds(j * L, L)]
                    out.append(accs[j] + (p0 + p1 * a + p2 * bb + p3 * (a * bb)))
                return tuple(out)

            accs = lax.fori_loop(
                0, CH, neuron,
                tuple(jnp.zeros((L,), jnp.float32) for _ in range(BATCH // L)))
            for j in range(BATCH // L):
                part_v[t, pl.ds(j * L, L)] = accs[j]
            out_descs.append(
                pltpu.async_copy(part_v.at[t], out_hbm.at[wrow[t]], sem_out))
        for dsc in out_descs:
            dsc.wait()

    return layer(x_t, idx_a, idx_b, pexp)


def _sc_layer(x_t, idx_a, idx_b, pexp):
    mesh = plsc.VectorSubcoreMesh(core_axis_name="c", subcore_axis_name="s")

    @functools.partial(
        pl.kernel,
        out_type=jax.ShapeDtypeStruct((PAD_OUT, BATCH), jnp.float32),
        mesh=mesh,
        scratch_types=[
            pltpu.VMEM((TPW, CH), jnp.int32),
            pltpu.VMEM((TPW, CH), jnp.int32),
            pltpu.VMEM((2, CH, BATCH), jnp.float32),
            pltpu.VMEM((2, CH, BATCH), jnp.float32),
            pltpu.VMEM((2, CH, BATCH), jnp.float32),
            pltpu.VMEM((2, CH, 4 * L), jnp.float32),
            pltpu.SemaphoreType.DMA,
            pltpu.SemaphoreType.DMA,
            pltpu.SemaphoreType.DMA,
            pltpu.SemaphoreType.DMA,
            pltpu.SemaphoreType.DMA,
        ],
    )
    def layer(x_hbm, ia_hbm, ib_hbm, p_hbm, out_hbm,
              ia_all, ib_all, ra_v, rb_v, o_v, p_v,
              sem_idx, sem_in0, sem_in1, sem_out0, sem_out1):
        wid = lax.axis_index("s") * NC + lax.axis_index("c")
        rbase = [jnp.minimum(t * NW + wid, NCHUNK - 1) * CH for t in range(TPW)]
        wbase = [(t * NW + wid) * CH for t in range(TPW)]
        sem_in = [sem_in0, sem_in1]
        sem_out = [sem_out0, sem_out1]

        idx_descs = []
        for t in range(TPW):
            idx_descs.append(
                pltpu.async_copy(ia_hbm.at[pl.ds(rbase[t], CH)], ia_all.at[t], sem_idx))
            idx_descs.append(
                pltpu.async_copy(ib_hbm.at[pl.ds(rbase[t], CH)], ib_all.at[t], sem_idx))
        for dsc in idx_descs:
            dsc.wait()

        in_descs = [None] * TPW
        out_descs = [None] * TPW

        def issue(t):
            b = t % 2
            in_descs[t] = (
                pltpu.async_copy(x_hbm.at[ia_all.at[t]], ra_v.at[b], sem_in[b]),
                pltpu.async_copy(x_hbm.at[ib_all.at[t]], rb_v.at[b], sem_in[b]),
                pltpu.async_copy(p_hbm.at[pl.ds(rbase[t], CH)], p_v.at[b], sem_in[b]),
            )

        issue(0)
        for t in range(TPW):
            b = t % 2
            for dsc in in_descs[t]:
                dsc.wait()
            if t + 1 < TPW:
                issue(t + 1)
            if t >= 2:
                out_descs[t - 2].wait()

            def neuron(i, carry):
                p0 = p_v[b, i, pl.ds(0, L)]
                p1 = p_v[b, i, pl.ds(L, L)]
                p2 = p_v[b, i, pl.ds(2 * L, L)]
                p3 = p_v[b, i, pl.ds(3 * L, L)]
                for j in range(BATCH // L):
                    a = ra_v[b, i, pl.ds(j * L, L)]
                    bb = rb_v[b, i, pl.ds(j * L, L)]
                    o_v[b, i, pl.ds(j * L, L)] = p0 + p1 * a + p2 * bb + p3 * (a * bb)
                return carry

            lax.fori_loop(0, CH, neuron, 0)
            out_descs[t] = pltpu.async_copy(
                o_v.at[b], out_hbm.at[pl.ds(wbase[t], CH)], sem_out[b])
        out_descs[TPW - 2].wait()
        out_descs[TPW - 1].wait()

    return layer(x_t, idx_a, idx_b, pexp)


_G = np.zeros((NW * TPW, NUM_CLASSES), dtype=np.float32)
_CPG = (OUT_DIM // NUM_CLASSES) // CH
for _c in range(NCHUNK):
    _G[_c, _c // _CPG] = 1.0


def _finish_body(p_ref, g_ref, o_ref):
    o_ref[...] = lax.dot_general(
        p_ref[...], g_ref[...], (((0,), (0,)), ((), ())),
        preferred_element_type=jnp.float32) / TAU


def _finish(partials):
    return pl.pallas_call(
        _finish_body,
        out_shape=jax.ShapeDtypeStruct((BATCH, NUM_CLASSES), jnp.float32),
    )(partials, jnp.asarray(_G))


def kernel(x, training, idx_a_0, idx_b_0, w_0, idx_a_1, idx_b_1, w_1,
           idx_a_2, idx_b_2, w_2, idx_a_3, idx_b_3, w_3):
    x = x.reshape((x.shape[0], -1))
    h, pexps = _coeffs(x, [w_0, w_1, w_2, w_3], training)
    idx_as = [idx_a_0, idx_a_1, idx_a_2, idx_a_3]
    idx_bs = [idx_b_0, idx_b_1, idx_b_2, idx_b_3]
    for l in range(3):
        h = _sc_layer(h, idx_as[l].astype(jnp.int32), idx_bs[l].astype(jnp.int32),
                      pexps[l])
    partials = _sc_layer_gsum(h, idx_as[3].astype(jnp.int32),
                              idx_bs[3].astype(jnp.int32), pexps[3])
    return _finish(partials)
